# baseline (speedup 1.0000x reference)
import jax
import jax.numpy as jnp
from jax.experimental import pallas as pl
from jax.experimental.pallas import tpu as pltpu


def _cls_body(xj_ref, w1_ref, b1_ref, w2_ref, b2_ref, w3_ref, b3_ref, out_ref):
    z = jnp.maximum(xj_ref[...] @ w1_ref[...] + b1_ref[...], 0.0)
    z = jnp.maximum(z @ w2_ref[...] + b2_ref[...], 0.0)
    out_ref[...] = z @ w3_ref[...] + b3_ref[...]


def _classifier(xj, params):
    b = xj.shape[0]
    return pl.pallas_call(
        _cls_body,
        out_shape=jax.ShapeDtypeStruct((b, 1), jnp.float32),
    )(xj, params["cls_w1"], params["cls_b1"][None, :],
      params["cls_w2"], params["cls_b2"][None, :],
      params["cls_w3"], params["cls_b3"][None, :])


def kernel(xd, xt, xt_edge_index, xt_batch, y, params):
    n = xt.shape[0]
    b = xd.shape[0]
    emb = jnp.take(params["emb_xd"], xd, axis=0)
    conv = jax.lax.conv_general_dilated(emb, params["conv_w"], window_strides=(1,),
                                        padding="VALID", dimension_numbers=("NCH", "OIH", "NCH"))
    conv = conv + params["conv_b"][None, :, None]
    xd_out = conv.reshape(b, 32 * 121) @ params["fc1_xd_w"] + params["fc1_xd_b"]
    src, dst = xt_edge_index[0], xt_edge_index[1]
    h = xt
    for gp, bp in zip(params["gin"], params["bn"]):
        agg = jax.ops.segment_sum(jnp.take(h, src, axis=0), dst, num_segments=n)
        z = h + agg
        z = jnp.maximum(z @ gp["w1"] + gp["b1"], 0.0)
        z = z @ gp["w2"] + gp["b2"]
        z = jnp.maximum(z, 0.0)
        h = (z / jnp.sqrt(1.0 + 1e-5)) * bp["g"] + bp["b"]
    pooled = jax.ops.segment_sum(h, xt_batch, num_segments=b)
    xt_out = jnp.maximum(pooled @ params["fc1_xt_w"] + params["fc1_xt_b"], 0.0)
    xj = jnp.concatenate([xd_out, xt_out], axis=1)
    out = _classifier(xj, params).squeeze(1)
    return (out, y)


# speedup vs baseline: 15.0900x; 15.0900x over previous
import functools

import jax
import jax.numpy as jnp
from jax import lax
from jax.experimental import pallas as pl
from jax.experimental.pallas import tpu as pltpu
from jax.experimental.pallas import tpu_sc as plsc

N = 100000
E = 1600000
B = 128
DIM = 32
HALF = 16

NC, NS, L = 2, 16, 16
W = 128
NCHUNK = E // W
CH_BASE = NCHUNK // NS
CH_REM = NCHUNK % NS
NBUF = 8
NGRP = -(-(CH_BASE + 1) // NBUF)
ZB = 256
DR = 6144
NZCH = DR // ZB
DR_LAST = N - (NS - 1) * DR
NZCH_LAST = DR_LAST // ZB
ZTAIL = DR_LAST - NZCH_LAST * ZB

_f32 = jnp.float32


def _agg_body(qlo, qhi, edges, out, acc, zbuf, idx, rows, isem, gsem, ssem):
    c = lax.axis_index("c")
    s = lax.axis_index("s")

    def _zrow(i, carry):
        zbuf[i, :] = jnp.zeros((L,), _f32)
        return carry
    lax.fori_loop(0, ZB, _zrow, 0)
    rbase = pl.multiple_of(s * DR, 8)

    for t in range(NZCH):
        pltpu.sync_copy(zbuf, acc.at[pl.ds(rbase + t * ZB, ZB)])

    @pl.when(s == NS - 1)
    def _():
        for t in range(NZCH, NZCH_LAST):
            pltpu.sync_copy(zbuf, acc.at[pl.ds(rbase + t * ZB, ZB)])
        pltpu.sync_copy(zbuf.at[pl.ds(0, ZTAIL)],
                        acc.at[pl.ds(rbase + NZCH_LAST * ZB, ZTAIL)])
    plsc.subcore_barrier()

    start = s * CH_BASE + jnp.minimum(s, CH_REM)
    end = start + CH_BASE + jnp.where(s < CH_REM, 1, 0)

    def _chunk_ops(b, ch):
        off = pl.multiple_of(ch * W, W)
        def idx_load():
            pltpu.async_copy(edges.at[:, pl.ds(off, W)], idx.at[b], isem.at[b])
        def gather_start():
            pltpu.make_async_copy(edges.at[:, pl.ds(off, W)], idx.at[b],
                                  isem.at[b]).wait()
            src = idx.at[b, 0]
            @pl.when(c == 0)
            def _():
                pltpu.async_copy(qlo.at[src], rows.at[b], gsem.at[b])
            @pl.when(c == 1)
            def _():
                pltpu.async_copy(qhi.at[src], rows.at[b], gsem.at[b])
        def gather_wait_scatter_start():
            pltpu.make_async_copy(qlo.at[idx.at[b, 0]], rows.at[b],
                                  gsem.at[b]).wait()
            pltpu.async_copy(rows.at[b], acc.at[idx.at[b, 1]], ssem.at[b],
                             add=True)
        def scatter_wait():
            pltpu.make_async_copy(rows.at[b], acc.at[idx.at[b, 1]],
                                  ssem.at[b]).wait()
        return idx_load, gather_start, gather_wait_scatter_start, scatter_wait

    def _group(g, carry):
        ch0 = start + g * NBUF
        ops = [_chunk_ops(b, ch0 + b) for b in range(NBUF)]
        for b in range(NBUF):
            @pl.when(g > 0)
            def _(b=b):
                ops[b][3]()
        for b in range(NBUF):
            @pl.when(ch0 + b < end)
            def _(b=b):
                ops[b][0]()
        for b in range(NBUF):
            @pl.when(ch0 + b < end)
            def _(b=b):
                ops[b][1]()
        for b in range(NBUF):
            @pl.when(ch0 + b < end)
            def _(b=b):
                ops[b][2]()
        return carry
    lax.fori_loop(0, NGRP, _group, 0)

    chf = start + (NGRP - 1) * NBUF
    for b in range(NBUF):
        ops = _chunk_ops(b, chf + b)
        @pl.when(chf + b < end)
        def _(sw=ops[3]):
            sw()

    plsc.subcore_barrier()

    for t in range(NZCH):
        r0 = pl.multiple_of(rbase + t * ZB, 8)
        pltpu.sync_copy(acc.at[pl.ds(r0, ZB)], zbuf)
        pltpu.sync_copy(zbuf, out.at[c, pl.ds(r0, ZB)])

    @pl.when(s == NS - 1)
    def _():
        for t in range(NZCH, NZCH_LAST):
            r0 = pl.multiple_of(rbase + t * ZB, 8)
            pltpu.sync_copy(acc.at[pl.ds(r0, ZB)], zbuf)
            pltpu.sync_copy(zbuf, out.at[c, pl.ds(r0, ZB)])
        r0 = pl.multiple_of(rbase + NZCH_LAST * ZB, 8)
        pltpu.sync_copy(acc.at[pl.ds(r0, ZTAIL)], zbuf.at[pl.ds(0, ZTAIL)])
        pltpu.sync_copy(zbuf.at[pl.ds(0, ZTAIL)], out.at[c, pl.ds(r0, ZTAIL)])


@functools.cache
def _make_agg():
  return pl.kernel(
    _agg_body,
    out_type=jax.ShapeDtypeStruct((NC, N, HALF), _f32),
    mesh=plsc.VectorSubcoreMesh(core_axis_name="c", subcore_axis_name="s",
                                num_cores=NC, num_subcores=NS),
    compiler_params=pltpu.CompilerParams(use_tc_tiling_on_sc=False),
    scratch_types=[
        pltpu.VMEM_SHARED((N, HALF), _f32),
        pltpu.VMEM((ZB, L), _f32),
        pltpu.VMEM((NBUF, 2, W), jnp.int32),
        pltpu.VMEM((NBUF, W, HALF), _f32),
        pltpu.SemaphoreType.DMA((NBUF,)),
        pltpu.SemaphoreType.DMA((NBUF,)),
        pltpu.SemaphoreType.DMA((NBUF,)),
    ],
  )


R = 2000
NBLK = N // R


def _pre_body(xt, w1, olo, ohi):
    q = jnp.dot(xt[...], w1[...], preferred_element_type=_f32)
    olo[...] = q[:, :HALF]
    ohi[...] = q[:, HALF:]


def _pre(xt, w1):
    return pl.pallas_call(
        _pre_body,
        grid=(NBLK,),
        in_specs=[
            pl.BlockSpec((R, 41), lambda i: (i, 0)),
            pl.BlockSpec((41, DIM), lambda i: (0, 0)),
        ],
        out_specs=[pl.BlockSpec((R, HALF), lambda i: (i, 0))] * 2,
        out_shape=[jax.ShapeDtypeStruct((N, HALF), _f32)] * 2,
    )(xt, w1)


def _layer_body(qlo, qhi, slo, shi, b1, w2, b2, gsc, gb, w1n, olo, ohi):
    q = jnp.concatenate([qlo[...], qhi[...]], axis=1)
    sagg = jnp.concatenate([slo[0], shi[0]], axis=1)
    z = jnp.maximum(q + sagg + b1[...], 0.0)
    z = jnp.maximum(jnp.dot(z, w2[...], preferred_element_type=_f32) + b2[...], 0.0)
    h = z * gsc[...] + gb[...]
    qn = jnp.dot(h, w1n[...], preferred_element_type=_f32)
    olo[...] = qn[:, :HALF]
    ohi[...] = qn[:, HALF:]


def _layer(qlo, qhi, s2, b1, w2, b2, gsc, gb, w1n):
    return pl.pallas_call(
        _layer_body,
        grid=(NBLK,),
        in_specs=[
            pl.BlockSpec((R, HALF), lambda i: (i, 0)),
            pl.BlockSpec((R, HALF), lambda i: (i, 0)),
            pl.BlockSpec((1, R, HALF), lambda i: (0, i, 0)),
            pl.BlockSpec((1, R, HALF), lambda i: (1, i, 0)),
            pl.BlockSpec((1, DIM), lambda i: (0, 0)),
            pl.BlockSpec((DIM, DIM), lambda i: (0, 0)),
            pl.BlockSpec((1, DIM), lambda i: (0, 0)),
            pl.BlockSpec((1, DIM), lambda i: (0, 0)),
            pl.BlockSpec((1, DIM), lambda i: (0, 0)),
            pl.BlockSpec((DIM, DIM), lambda i: (0, 0)),
        ],
        out_specs=[pl.BlockSpec((R, HALF), lambda i: (i, 0))] * 2,
        out_shape=[jax.ShapeDtypeStruct((N, HALF), _f32)] * 2,
    )(qlo, qhi, s2, s2, b1, w2, b2, gsc, gb, w1n)


def _last_body(qlo, qhi, slo, shi, b1, w2, b2, gsc, gb, oh):
    q = jnp.concatenate([qlo[...], qhi[...]], axis=1)
    sagg = jnp.concatenate([slo[0], shi[0]], axis=1)
    z = jnp.maximum(q + sagg + b1[...], 0.0)
    z = jnp.maximum(jnp.dot(z, w2[...], preferred_element_type=_f32) + b2[...], 0.0)
    oh[...] = z * gsc[...] + gb[...]


def _last(qlo, qhi, s2, b1, w2, b2, gsc, gb):
    return pl.pallas_call(
        _last_body,
        grid=(NBLK,),
        in_specs=[
            pl.BlockSpec((R, HALF), lambda i: (i, 0)),
            pl.BlockSpec((R, HALF), lambda i: (i, 0)),
            pl.BlockSpec((1, R, HALF), lambda i: (0, i, 0)),
            pl.BlockSpec((1, R, HALF), lambda i: (1, i, 0)),
            pl.BlockSpec((1, DIM), lambda i: (0, 0)),
            pl.BlockSpec((DIM, DIM), lambda i: (0, 0)),
            pl.BlockSpec((1, DIM), lambda i: (0, 0)),
            pl.BlockSpec((1, DIM), lambda i: (0, 0)),
            pl.BlockSpec((1, DIM), lambda i: (0, 0)),
        ],
        out_specs=pl.BlockSpec((R, DIM), lambda i: (i, 0)),
        out_shape=jax.ShapeDtypeStruct((N, DIM), _f32),
    )(qlo, qhi, s2, s2, b1, w2, b2, gsc, gb)


def _emb_body(xdF, table, out):
    ohot = (xdF[...] == lax.broadcasted_iota(jnp.int32, (100 * B, 65), 1)).astype(_f32)
    out[...] = jnp.dot(ohot, table[...], preferred_element_type=_f32)


def _mm_body(a, b, out):
    out[...] = jnp.dot(a[...], b[...], preferred_element_type=_f32)


def _conv_slices_body(p3, cb, out):
    acc = p3[:, 0:121, 0:DIM]
    for k in range(1, 8):
        acc = acc + p3[:, k:k + 121, k * DIM:(k + 1) * DIM]
    out[...] = acc + cb[...].reshape(1, 1, DIM)


def _mm_bias_body(a, b, bias, out):
    out[...] = jnp.dot(a[...], b[...], preferred_element_type=_f32) + bias[...]


def _pc(body, out_shape, *args):
    return pl.pallas_call(body, out_shape=out_shape)(*args)


def _drug(xd, table, wr2, cb, wperm, fb):
    xdF = xd.T.reshape(100 * B, 1)
    emb3 = _pc(_emb_body, jax.ShapeDtypeStruct((100 * B, 128), _f32), xdF, table)
    at = emb3.reshape(100, B * 128).T
    p2 = _pc(_mm_body, jax.ShapeDtypeStruct((B * 128, 8 * DIM), _f32), at, wr2)
    p3 = p2.reshape(B, 128, 8 * DIM)
    bb = 16
    acc = pl.pallas_call(
        _conv_slices_body,
        grid=(B // bb,),
        in_specs=[
            pl.BlockSpec((bb, 128, 8 * DIM), lambda i: (i, 0, 0)),
            pl.BlockSpec((1, DIM), lambda i: (0, 0)),
        ],
        out_specs=pl.BlockSpec((bb, 121, DIM), lambda i: (i, 0, 0)),
        out_shape=jax.ShapeDtypeStruct((B, 121, DIM), _f32),
    )(p3, cb)
    flat = acc.reshape(B, 121 * DIM)
    return _pc(_mm_bias_body, jax.ShapeDtypeStruct((B, 128), _f32), flat, wperm, fb)


def _final_body(h, bt, xdo, fw, fb, w1, b1, w2, b2, w3, b3, pooled, out):
    i = pl.program_id(0)

    @pl.when(i == 0)
    def _():
        pooled[...] = jnp.zeros_like(pooled)

    bb = bt[0, 0, :]
    ohot = (bb[:, None] == lax.broadcasted_iota(jnp.int32, (R, B), 1)).astype(_f32)
    pooled[...] += lax.dot_general(ohot, h[...], (((0,), (0,)), ((), ())),
                                   preferred_element_type=_f32)

    @pl.when(i == NBLK - 1)
    def _():
        xt_out = jnp.maximum(
            jnp.dot(pooled[...], fw[...], preferred_element_type=_f32) + fb[...], 0.0)
        xj = jnp.concatenate([xdo[...], xt_out], axis=1)
        z = jnp.maximum(jnp.dot(xj, w1[...], preferred_element_type=_f32) + b1[...], 0.0)
        z = jnp.maximum(jnp.dot(z, w2[...], preferred_element_type=_f32) + b2[...], 0.0)
        out[...] = jnp.dot(z, w3[...], preferred_element_type=_f32) + b3[...]


def _final(h, batch3, xdo, fw, fb, w1, b1, w2, b2, w3, b3):
    pooled, out = pl.pallas_call(
        _final_body,
        grid=(NBLK,),
        in_specs=[
            pl.BlockSpec((R, DIM), lambda i: (i, 0)),
            pl.BlockSpec((1, 1, R), lambda i: (i, 0, 0)),
            pl.BlockSpec((B, 128), lambda i: (0, 0)),
            pl.BlockSpec((DIM, 128), lambda i: (0, 0)),
            pl.BlockSpec((1, 128), lambda i: (0, 0)),
            pl.BlockSpec((256, 1024), lambda i: (0, 0)),
            pl.BlockSpec((1, 1024), lambda i: (0, 0)),
            pl.BlockSpec((1024, 256), lambda i: (0, 0)),
            pl.BlockSpec((1, 256), lambda i: (0, 0)),
            pl.BlockSpec((256, 1), lambda i: (0, 0)),
            pl.BlockSpec((1, 1), lambda i: (0, 0)),
        ],
        out_specs=[
            pl.BlockSpec((B, DIM), lambda i: (0, 0)),
            pl.BlockSpec((B, 1), lambda i: (0, 0)),
        ],
        out_shape=[
            jax.ShapeDtypeStruct((B, DIM), _f32),
            jax.ShapeDtypeStruct((B, 1), _f32),
        ],
    )(h, batch3, xdo, fw, fb, w1, b1, w2, b2, w3, b3)
    return out


def kernel(xd, xt, xt_edge_index, xt_batch, y, params):
    p = params
    bn_scale = jnp.float32(1.0 / jnp.sqrt(1.0 + 1e-5))

    wr2 = p["conv_w"].transpose(1, 2, 0).reshape(100, 8 * DIM)
    wperm = p["fc1_xd_w"].reshape(DIM, 121, 128).transpose(1, 0, 2).reshape(121 * DIM, 128)
    xd_out = _drug(xd, p["emb_xd"], wr2, p["conv_b"].reshape(1, DIM), wperm,
                   p["fc1_xd_b"].reshape(1, 128))

    edges = xt_edge_index.astype(jnp.int32)
    qlo, qhi = _pre(xt, p["gin"][0]["w1"])
    h = None
    agg = _make_agg()
    for k in range(5):
        s2 = agg(qlo, qhi, edges)
        gp, bp = p["gin"][k], p["bn"][k]
        b1 = gp["b1"].reshape(1, DIM)
        w2, b2 = gp["w2"], gp["b2"].reshape(1, DIM)
        gsc = (bp["g"] * bn_scale).reshape(1, DIM)
        gb = bp["b"].reshape(1, DIM)
        if k < 4:
            qlo, qhi = _layer(qlo, qhi, s2, b1, w2, b2, gsc, gb,
                              p["gin"][k + 1]["w1"])
        else:
            h = _last(qlo, qhi, s2, b1, w2, b2, gsc, gb)

    batch3 = xt_batch.astype(jnp.int32).reshape(NBLK, 1, R)
    out = _final(h, batch3, xd_out, p["fc1_xt_w"], p["fc1_xt_b"].reshape(1, 128),
                 p["cls_w1"], p["cls_b1"].reshape(1, 1024),
                 p["cls_w2"], p["cls_b2"].reshape(1, 256),
                 p["cls_w3"], p["cls_b3"].reshape(1, 1))
    return (out.reshape(B), y)


# speedup vs baseline: 15.4765x; 1.0256x over previous
import functools

import jax
import jax.numpy as jnp
from jax import lax
from jax.experimental import pallas as pl
from jax.experimental.pallas import tpu as pltpu
from jax.experimental.pallas import tpu_sc as plsc

N = 100000
E = 1600000
B = 128
DIM = 32
HALF = 16

NC, NS, L = 2, 16, 16
W = 128
SCW = 256
NSUP = E // SCW
SUP_BASE = NSUP // NS
SUP_REM = NSUP % NS
NBUF = 5
NGRP = -(-(SUP_BASE + 1) // NBUF)
ZB = 128
DR = 6144
NZCH = DR // ZB
DR_LAST = N - (NS - 1) * DR
NZCH_LAST = DR_LAST // ZB
ZTAIL = DR_LAST - NZCH_LAST * ZB

_f32 = jnp.float32


def _agg_body(qlo, qhi, edges, out, acc, zbuf, idx, rows, isem, gsem, ssem):
    c = lax.axis_index("c")
    s = lax.axis_index("s")

    def _zrow(i, carry):
        zbuf[i, :] = jnp.zeros((L,), _f32)
        return carry
    lax.fori_loop(0, ZB, _zrow, 0)
    rbase = pl.multiple_of(s * DR, 8)

    for t in range(NZCH):
        pltpu.sync_copy(zbuf, acc.at[pl.ds(rbase + t * ZB, ZB)])

    @pl.when(s == NS - 1)
    def _():
        for t in range(NZCH, NZCH_LAST):
            pltpu.sync_copy(zbuf, acc.at[pl.ds(rbase + t * ZB, ZB)])
        pltpu.sync_copy(zbuf.at[pl.ds(0, ZTAIL)],
                        acc.at[pl.ds(rbase + NZCH_LAST * ZB, ZTAIL)])
    plsc.subcore_barrier()

    start = s * SUP_BASE + jnp.minimum(s, SUP_REM)
    end = start + SUP_BASE + jnp.where(s < SUP_REM, 1, 0)

    def _chunk_ops(b, ch):
        off = pl.multiple_of(ch * SCW, SCW)
        def idx_load():
            pltpu.async_copy(edges.at[:, pl.ds(off, SCW)], idx.at[b], isem.at[b])
        def gather_start():
            pltpu.make_async_copy(edges.at[:, pl.ds(off, SCW)], idx.at[b],
                                  isem.at[b]).wait()
            src = idx.at[b, 0]
            @pl.when(c == 0)
            def _():
                pltpu.async_copy(qlo.at[src], rows.at[b], gsem.at[b])
            @pl.when(c == 1)
            def _():
                pltpu.async_copy(qhi.at[src], rows.at[b], gsem.at[b])
        def gather_wait_scatter_start():
            pltpu.make_async_copy(qlo.at[idx.at[b, 0]], rows.at[b],
                                  gsem.at[b]).wait()
            pltpu.async_copy(rows.at[b], acc.at[idx.at[b, 1]], ssem.at[b],
                             add=True)
        def scatter_wait():
            pltpu.make_async_copy(rows.at[b], acc.at[idx.at[b, 1]],
                                  ssem.at[b]).wait()
        return idx_load, gather_start, gather_wait_scatter_start, scatter_wait

    def _group(g, carry):
        ch0 = start + g * NBUF
        ops = [_chunk_ops(b, ch0 + b) for b in range(NBUF)]
        for b in range(NBUF):
            @pl.when(g > 0)
            def _(b=b):
                ops[b][3]()
        for b in range(NBUF):
            @pl.when(ch0 + b < end)
            def _(b=b):
                ops[b][0]()
        for b in range(NBUF):
            @pl.when(ch0 + b < end)
            def _(b=b):
                ops[b][1]()
        for b in range(NBUF):
            @pl.when(ch0 + b < end)
            def _(b=b):
                ops[b][2]()
        return carry
    lax.fori_loop(0, NGRP, _group, 0)

    chf = start + (NGRP - 1) * NBUF
    for b in range(NBUF):
        ops = _chunk_ops(b, chf + b)
        @pl.when(chf + b < end)
        def _(sw=ops[3]):
            sw()

    plsc.subcore_barrier()

    @pl.when(s < NS - 1)
    def _():
        pltpu.sync_copy(acc.at[pl.ds(rbase, DR)], out.at[c, pl.ds(rbase, DR)])

    @pl.when(s == NS - 1)
    def _():
        pltpu.sync_copy(acc.at[pl.ds(rbase, DR_LAST)],
                        out.at[c, pl.ds(rbase, DR_LAST)])


@functools.cache
def _make_agg():
  return pl.kernel(
    _agg_body,
    out_type=jax.ShapeDtypeStruct((NC, N, HALF), _f32),
    mesh=plsc.VectorSubcoreMesh(core_axis_name="c", subcore_axis_name="s",
                                num_cores=NC, num_subcores=NS),
    compiler_params=pltpu.CompilerParams(use_tc_tiling_on_sc=False),
    scratch_types=[
        pltpu.VMEM_SHARED((N, HALF), _f32),
        pltpu.VMEM((ZB, L), _f32),
        pltpu.VMEM((NBUF, 2, SCW), jnp.int32),
        pltpu.VMEM((NBUF, SCW, HALF), _f32),
        pltpu.SemaphoreType.DMA((NBUF,)),
        pltpu.SemaphoreType.DMA((NBUF,)),
        pltpu.SemaphoreType.DMA((NBUF,)),
    ],
  )


R = 2000
NBLK = N // R


def _pre_body(xt, w1, olo, ohi):
    q = jnp.dot(xt[...], w1[...], preferred_element_type=_f32)
    olo[...] = q[:, :HALF]
    ohi[...] = q[:, HALF:]


def _pre(xt, w1):
    return pl.pallas_call(
        _pre_body,
        grid=(NBLK,),
        in_specs=[
            pl.BlockSpec((R, 41), lambda i: (i, 0)),
            pl.BlockSpec((41, DIM), lambda i: (0, 0)),
        ],
        out_specs=[pl.BlockSpec((R, HALF), lambda i: (i, 0))] * 2,
        out_shape=[jax.ShapeDtypeStruct((N, HALF), _f32)] * 2,
    )(xt, w1)


def _layer_body(qlo, qhi, slo, shi, b1, w2, b2, gsc, gb, w1n, olo, ohi):
    q = jnp.concatenate([qlo[...], qhi[...]], axis=1)
    sagg = jnp.concatenate([slo[0], shi[0]], axis=1)
    z = jnp.maximum(q + sagg + b1[...], 0.0)
    z = jnp.maximum(jnp.dot(z, w2[...], preferred_element_type=_f32) + b2[...], 0.0)
    h = z * gsc[...] + gb[...]
    qn = jnp.dot(h, w1n[...], preferred_element_type=_f32)
    olo[...] = qn[:, :HALF]
    ohi[...] = qn[:, HALF:]


def _layer(qlo, qhi, s2, b1, w2, b2, gsc, gb, w1n):
    return pl.pallas_call(
        _layer_body,
        grid=(NBLK,),
        in_specs=[
            pl.BlockSpec((R, HALF), lambda i: (i, 0)),
            pl.BlockSpec((R, HALF), lambda i: (i, 0)),
            pl.BlockSpec((1, R, HALF), lambda i: (0, i, 0)),
            pl.BlockSpec((1, R, HALF), lambda i: (1, i, 0)),
            pl.BlockSpec((1, DIM), lambda i: (0, 0)),
            pl.BlockSpec((DIM, DIM), lambda i: (0, 0)),
            pl.BlockSpec((1, DIM), lambda i: (0, 0)),
            pl.BlockSpec((1, DIM), lambda i: (0, 0)),
            pl.BlockSpec((1, DIM), lambda i: (0, 0)),
            pl.BlockSpec((DIM, DIM), lambda i: (0, 0)),
        ],
        out_specs=[pl.BlockSpec((R, HALF), lambda i: (i, 0))] * 2,
        out_shape=[jax.ShapeDtypeStruct((N, HALF), _f32)] * 2,
    )(qlo, qhi, s2, s2, b1, w2, b2, gsc, gb, w1n)


def _last_body(qlo, qhi, slo, shi, b1, w2, b2, gsc, gb, oh):
    q = jnp.concatenate([qlo[...], qhi[...]], axis=1)
    sagg = jnp.concatenate([slo[0], shi[0]], axis=1)
    z = jnp.maximum(q + sagg + b1[...], 0.0)
    z = jnp.maximum(jnp.dot(z, w2[...], preferred_element_type=_f32) + b2[...], 0.0)
    oh[...] = z * gsc[...] + gb[...]


def _last(qlo, qhi, s2, b1, w2, b2, gsc, gb):
    return pl.pallas_call(
        _last_body,
        grid=(NBLK,),
        in_specs=[
            pl.BlockSpec((R, HALF), lambda i: (i, 0)),
            pl.BlockSpec((R, HALF), lambda i: (i, 0)),
            pl.BlockSpec((1, R, HALF), lambda i: (0, i, 0)),
            pl.BlockSpec((1, R, HALF), lambda i: (1, i, 0)),
            pl.BlockSpec((1, DIM), lambda i: (0, 0)),
            pl.BlockSpec((DIM, DIM), lambda i: (0, 0)),
            pl.BlockSpec((1, DIM), lambda i: (0, 0)),
            pl.BlockSpec((1, DIM), lambda i: (0, 0)),
            pl.BlockSpec((1, DIM), lambda i: (0, 0)),
        ],
        out_specs=pl.BlockSpec((R, DIM), lambda i: (i, 0)),
        out_shape=jax.ShapeDtypeStruct((N, DIM), _f32),
    )(qlo, qhi, s2, s2, b1, w2, b2, gsc, gb)


def _emb_body(xdF, table, out):
    ohot = (xdF[...] == lax.broadcasted_iota(jnp.int32, (100 * B, 65), 1)).astype(_f32)
    out[...] = jnp.dot(ohot, table[...], preferred_element_type=_f32)


def _mm_body(a, b, out):
    out[...] = jnp.dot(a[...], b[...], preferred_element_type=_f32)


def _conv_slices_body(p3, cb, out):
    acc = p3[:, 0:121, 0:DIM]
    for k in range(1, 8):
        acc = acc + p3[:, k:k + 121, k * DIM:(k + 1) * DIM]
    out[...] = acc + cb[...].reshape(1, 1, DIM)


def _mm_bias_body(a, b, bias, out):
    out[...] = jnp.dot(a[...], b[...], preferred_element_type=_f32) + bias[...]


def _pc(body, out_shape, *args):
    return pl.pallas_call(body, out_shape=out_shape)(*args)


def _drug(xd, table, wr2, cb, wperm, fb):
    xdF = xd.T.reshape(100 * B, 1)
    emb3 = _pc(_emb_body, jax.ShapeDtypeStruct((100 * B, 128), _f32), xdF, table)
    at = emb3.reshape(100, B * 128).T
    p2 = _pc(_mm_body, jax.ShapeDtypeStruct((B * 128, 8 * DIM), _f32), at, wr2)
    p3 = p2.reshape(B, 128, 8 * DIM)
    bb = 16
    acc = pl.pallas_call(
        _conv_slices_body,
        grid=(B // bb,),
        in_specs=[
            pl.BlockSpec((bb, 128, 8 * DIM), lambda i: (i, 0, 0)),
            pl.BlockSpec((1, DIM), lambda i: (0, 0)),
        ],
        out_specs=pl.BlockSpec((bb, 121, DIM), lambda i: (i, 0, 0)),
        out_shape=jax.ShapeDtypeStruct((B, 121, DIM), _f32),
    )(p3, cb)
    flat = acc.reshape(B, 121 * DIM)
    return _pc(_mm_bias_body, jax.ShapeDtypeStruct((B, 128), _f32), flat, wperm, fb)


def _final_body(h, bt, xdo, fw, fb, w1, b1, w2, b2, w3, b3, pooled, out):
    i = pl.program_id(0)

    @pl.when(i == 0)
    def _():
        pooled[...] = jnp.zeros_like(pooled)

    bb = bt[0, 0, :]
    ohot = (bb[:, None] == lax.broadcasted_iota(jnp.int32, (R, B), 1)).astype(_f32)
    pooled[...] += lax.dot_general(ohot, h[...], (((0,), (0,)), ((), ())),
                                   preferred_element_type=_f32)

    @pl.when(i == NBLK - 1)
    def _():
        xt_out = jnp.maximum(
            jnp.dot(pooled[...], fw[...], preferred_element_type=_f32) + fb[...], 0.0)
        xj = jnp.concatenate([xdo[...], xt_out], axis=1)
        z = jnp.maximum(jnp.dot(xj, w1[...], preferred_element_type=_f32) + b1[...], 0.0)
        z = jnp.maximum(jnp.dot(z, w2[...], preferred_element_type=_f32) + b2[...], 0.0)
        out[...] = jnp.dot(z, w3[...], preferred_element_type=_f32) + b3[...]


def _final(h, batch3, xdo, fw, fb, w1, b1, w2, b2, w3, b3):
    pooled, out = pl.pallas_call(
        _final_body,
        grid=(NBLK,),
        in_specs=[
            pl.BlockSpec((R, DIM), lambda i: (i, 0)),
            pl.BlockSpec((1, 1, R), lambda i: (i, 0, 0)),
            pl.BlockSpec((B, 128), lambda i: (0, 0)),
            pl.BlockSpec((DIM, 128), lambda i: (0, 0)),
            pl.BlockSpec((1, 128), lambda i: (0, 0)),
            pl.BlockSpec((256, 1024), lambda i: (0, 0)),
            pl.BlockSpec((1, 1024), lambda i: (0, 0)),
            pl.BlockSpec((1024, 256), lambda i: (0, 0)),
            pl.BlockSpec((1, 256), lambda i: (0, 0)),
            pl.BlockSpec((256, 1), lambda i: (0, 0)),
            pl.BlockSpec((1, 1), lambda i: (0, 0)),
        ],
        out_specs=[
            pl.BlockSpec((B, DIM), lambda i: (0, 0)),
            pl.BlockSpec((B, 1), lambda i: (0, 0)),
        ],
        out_shape=[
            jax.ShapeDtypeStruct((B, DIM), _f32),
            jax.ShapeDtypeStruct((B, 1), _f32),
        ],
    )(h, batch3, xdo, fw, fb, w1, b1, w2, b2, w3, b3)
    return out


def kernel(xd, xt, xt_edge_index, xt_batch, y, params):
    p = params
    bn_scale = jnp.float32(1.0 / jnp.sqrt(1.0 + 1e-5))

    wr2 = p["conv_w"].transpose(1, 2, 0).reshape(100, 8 * DIM)
    wperm = p["fc1_xd_w"].reshape(DIM, 121, 128).transpose(1, 0, 2).reshape(121 * DIM, 128)
    xd_out = _drug(xd, p["emb_xd"], wr2, p["conv_b"].reshape(1, DIM), wperm,
                   p["fc1_xd_b"].reshape(1, 128))

    edges = xt_edge_index.astype(jnp.int32)
    qlo, qhi = _pre(xt, p["gin"][0]["w1"])
    h = None
    agg = _make_agg()
    for k in range(5):
        s2 = agg(qlo, qhi, edges)
        gp, bp = p["gin"][k], p["bn"][k]
        b1 = gp["b1"].reshape(1, DIM)
        w2, b2 = gp["w2"], gp["b2"].reshape(1, DIM)
        gsc = (bp["g"] * bn_scale).reshape(1, DIM)
        gb = bp["b"].reshape(1, DIM)
        if k < 4:
            qlo, qhi = _layer(qlo, qhi, s2, b1, w2, b2, gsc, gb,
                              p["gin"][k + 1]["w1"])
        else:
            h = _last(qlo, qhi, s2, b1, w2, b2, gsc, gb)

    batch3 = xt_batch.astype(jnp.int32).reshape(NBLK, 1, R)
    out = _final(h, batch3, xd_out, p["fc1_xt_w"], p["fc1_xt_b"].reshape(1, 128),
                 p["cls_w1"], p["cls_b1"].reshape(1, 1024),
                 p["cls_w2"], p["cls_b2"].reshape(1, 256),
                 p["cls_w3"], p["cls_b3"].reshape(1, 1))
    return (out.reshape(B), y)


# speedup vs baseline: 15.9134x; 1.0282x over previous
import functools

import jax
import jax.numpy as jnp
from jax import lax
from jax.experimental import pallas as pl
from jax.experimental.pallas import tpu as pltpu
from jax.experimental.pallas import tpu_sc as plsc

N = 100000
E = 1600000
B = 128
DIM = 32
HALF = 16

NC, NS, L = 2, 16, 16
W = 128
SCW = 256
NSUP = E // SCW
SUP_BASE = NSUP // NS
SUP_REM = NSUP % NS
NBUF = 5
NGRP = -(-(SUP_BASE + 1) // NBUF)
ZB = 128
DR = 6144
NZCH = DR // ZB
DR_LAST = N - (NS - 1) * DR
NZCH_LAST = DR_LAST // ZB
ZTAIL = DR_LAST - NZCH_LAST * ZB

_f32 = jnp.float32


def _agg_body(qlo, qhi, edges, out, acc, zbuf, idx, rows, isem, gsem, ssem):
    c = lax.axis_index("c")
    s = lax.axis_index("s")

    def _zrow(i, carry):
        zbuf[i, :] = jnp.zeros((L,), _f32)
        return carry
    lax.fori_loop(0, ZB, _zrow, 0)
    rbase = pl.multiple_of(s * DR, 8)

    for t in range(NZCH):
        pltpu.sync_copy(zbuf, acc.at[pl.ds(rbase + t * ZB, ZB)])

    @pl.when(s == NS - 1)
    def _():
        for t in range(NZCH, NZCH_LAST):
            pltpu.sync_copy(zbuf, acc.at[pl.ds(rbase + t * ZB, ZB)])
        pltpu.sync_copy(zbuf.at[pl.ds(0, ZTAIL)],
                        acc.at[pl.ds(rbase + NZCH_LAST * ZB, ZTAIL)])
    plsc.subcore_barrier()

    start = s * SUP_BASE + jnp.minimum(s, SUP_REM)
    end = start + SUP_BASE + jnp.where(s < SUP_REM, 1, 0)

    def _chunk_ops(b, ch):
        off = pl.multiple_of(ch * SCW, SCW)
        def idx_load():
            pltpu.async_copy(edges.at[:, pl.ds(off, SCW)], idx.at[b], isem.at[b])
        def gather_start():
            pltpu.make_async_copy(edges.at[:, pl.ds(off, SCW)], idx.at[b],
                                  isem.at[b]).wait()
            src = idx.at[b, 0]
            @pl.when(c == 0)
            def _():
                pltpu.async_copy(qlo.at[src], rows.at[b], gsem.at[b])
            @pl.when(c == 1)
            def _():
                pltpu.async_copy(qhi.at[src], rows.at[b], gsem.at[b])
        def gather_wait_scatter_start():
            pltpu.make_async_copy(qlo.at[idx.at[b, 0]], rows.at[b],
                                  gsem.at[b]).wait()
            pltpu.async_copy(rows.at[b], acc.at[idx.at[b, 1]], ssem.at[b],
                             add=True)
        def scatter_wait():
            pltpu.make_async_copy(rows.at[b], acc.at[idx.at[b, 1]],
                                  ssem.at[b]).wait()
        return idx_load, gather_start, gather_wait_scatter_start, scatter_wait

    def _group(g, carry):
        ch0 = start + g * NBUF
        ops = [_chunk_ops(b, ch0 + b) for b in range(NBUF)]
        for b in range(NBUF):
            @pl.when(g > 0)
            def _(b=b):
                ops[b][3]()
        for b in range(NBUF):
            @pl.when(ch0 + b < end)
            def _(b=b):
                ops[b][0]()
        for b in range(NBUF):
            @pl.when(ch0 + b < end)
            def _(b=b):
                ops[b][1]()
        for b in range(NBUF):
            @pl.when(ch0 + b < end)
            def _(b=b):
                ops[b][2]()
        return carry
    lax.fori_loop(0, NGRP, _group, 0)

    chf = start + (NGRP - 1) * NBUF
    for b in range(NBUF):
        ops = _chunk_ops(b, chf + b)
        @pl.when(chf + b < end)
        def _(sw=ops[3]):
            sw()

    plsc.subcore_barrier()

    @pl.when(s < NS - 1)
    def _():
        pltpu.sync_copy(acc.at[pl.ds(rbase, DR)], out.at[c, pl.ds(rbase, DR)])

    @pl.when(s == NS - 1)
    def _():
        pltpu.sync_copy(acc.at[pl.ds(rbase, DR_LAST)],
                        out.at[c, pl.ds(rbase, DR_LAST)])


@functools.cache
def _make_agg():
  return pl.kernel(
    _agg_body,
    out_type=jax.ShapeDtypeStruct((NC, N, HALF), _f32),
    mesh=plsc.VectorSubcoreMesh(core_axis_name="c", subcore_axis_name="s",
                                num_cores=NC, num_subcores=NS),
    compiler_params=pltpu.CompilerParams(use_tc_tiling_on_sc=False),
    scratch_types=[
        pltpu.VMEM_SHARED((N, HALF), _f32),
        pltpu.VMEM((ZB, L), _f32),
        pltpu.VMEM((NBUF, 2, SCW), jnp.int32),
        pltpu.VMEM((NBUF, SCW, HALF), _f32),
        pltpu.SemaphoreType.DMA((NBUF,)),
        pltpu.SemaphoreType.DMA((NBUF,)),
        pltpu.SemaphoreType.DMA((NBUF,)),
    ],
  )


R = 5000
NBLK = N // R


def _pre_body(xt, w1, olo, ohi):
    q = jnp.dot(xt[...], w1[...], preferred_element_type=_f32)
    olo[...] = q[:, :HALF]
    ohi[...] = q[:, HALF:]


def _pre(xt, w1):
    return pl.pallas_call(
        _pre_body,
        grid=(NBLK,),
        in_specs=[
            pl.BlockSpec((R, 41), lambda i: (i, 0)),
            pl.BlockSpec((41, DIM), lambda i: (0, 0)),
        ],
        out_specs=[pl.BlockSpec((R, HALF), lambda i: (i, 0))] * 2,
        out_shape=[jax.ShapeDtypeStruct((N, HALF), _f32)] * 2,
    )(xt, w1)


def _layer_body(qlo, qhi, slo, shi, b1, w2, b2, gsc, gb, w1n, olo, ohi):
    q = jnp.concatenate([qlo[...], qhi[...]], axis=1)
    sagg = jnp.concatenate([slo[0], shi[0]], axis=1)
    z = jnp.maximum(q + sagg + b1[...], 0.0)
    z = jnp.maximum(jnp.dot(z, w2[...], preferred_element_type=_f32) + b2[...], 0.0)
    h = z * gsc[...] + gb[...]
    qn = jnp.dot(h, w1n[...], preferred_element_type=_f32)
    olo[...] = qn[:, :HALF]
    ohi[...] = qn[:, HALF:]


def _layer(qlo, qhi, s2, b1, w2, b2, gsc, gb, w1n):
    return pl.pallas_call(
        _layer_body,
        grid=(NBLK,),
        in_specs=[
            pl.BlockSpec((R, HALF), lambda i: (i, 0)),
            pl.BlockSpec((R, HALF), lambda i: (i, 0)),
            pl.BlockSpec((1, R, HALF), lambda i: (0, i, 0)),
            pl.BlockSpec((1, R, HALF), lambda i: (1, i, 0)),
            pl.BlockSpec((1, DIM), lambda i: (0, 0)),
            pl.BlockSpec((DIM, DIM), lambda i: (0, 0)),
            pl.BlockSpec((1, DIM), lambda i: (0, 0)),
            pl.BlockSpec((1, DIM), lambda i: (0, 0)),
            pl.BlockSpec((1, DIM), lambda i: (0, 0)),
            pl.BlockSpec((DIM, DIM), lambda i: (0, 0)),
        ],
        out_specs=[pl.BlockSpec((R, HALF), lambda i: (i, 0))] * 2,
        out_shape=[jax.ShapeDtypeStruct((N, HALF), _f32)] * 2,
    )(qlo, qhi, s2, s2, b1, w2, b2, gsc, gb, w1n)


def _last_body(qlo, qhi, slo, shi, b1, w2, b2, gsc, gb, oh):
    q = jnp.concatenate([qlo[...], qhi[...]], axis=1)
    sagg = jnp.concatenate([slo[0], shi[0]], axis=1)
    z = jnp.maximum(q + sagg + b1[...], 0.0)
    z = jnp.maximum(jnp.dot(z, w2[...], preferred_element_type=_f32) + b2[...], 0.0)
    oh[...] = z * gsc[...] + gb[...]


def _last(qlo, qhi, s2, b1, w2, b2, gsc, gb):
    return pl.pallas_call(
        _last_body,
        grid=(NBLK,),
        in_specs=[
            pl.BlockSpec((R, HALF), lambda i: (i, 0)),
            pl.BlockSpec((R, HALF), lambda i: (i, 0)),
            pl.BlockSpec((1, R, HALF), lambda i: (0, i, 0)),
            pl.BlockSpec((1, R, HALF), lambda i: (1, i, 0)),
            pl.BlockSpec((1, DIM), lambda i: (0, 0)),
            pl.BlockSpec((DIM, DIM), lambda i: (0, 0)),
            pl.BlockSpec((1, DIM), lambda i: (0, 0)),
            pl.BlockSpec((1, DIM), lambda i: (0, 0)),
            pl.BlockSpec((1, DIM), lambda i: (0, 0)),
        ],
        out_specs=pl.BlockSpec((R, DIM), lambda i: (i, 0)),
        out_shape=jax.ShapeDtypeStruct((N, DIM), _f32),
    )(qlo, qhi, s2, s2, b1, w2, b2, gsc, gb)


def _emb_body(xdF, table, out):
    ohot = (xdF[...] == lax.broadcasted_iota(jnp.int32, (100 * B, 65), 1)).astype(_f32)
    out[...] = jnp.dot(ohot, table[...], preferred_element_type=_f32)


def _mm_body(a, b, out):
    out[...] = jnp.dot(a[...], b[...], preferred_element_type=_f32)


def _conv_slices_body(p3, cb, out):
    acc = p3[:, 0:121, 0:DIM]
    for k in range(1, 8):
        acc = acc + p3[:, k:k + 121, k * DIM:(k + 1) * DIM]
    out[...] = acc + cb[...].reshape(1, 1, DIM)


def _mm_bias_body(a, b, bias, out):
    out[...] = jnp.dot(a[...], b[...], preferred_element_type=_f32) + bias[...]


def _pc(body, out_shape, *args):
    return pl.pallas_call(body, out_shape=out_shape)(*args)


def _drug(xd, table, wr2, cb, wperm, fb):
    xdF = xd.T.reshape(100 * B, 1)
    emb3 = _pc(_emb_body, jax.ShapeDtypeStruct((100 * B, 128), _f32), xdF, table)
    at = emb3.reshape(100, B * 128).T
    p2 = _pc(_mm_body, jax.ShapeDtypeStruct((B * 128, 8 * DIM), _f32), at, wr2)
    p3 = p2.reshape(B, 128, 8 * DIM)
    bb = 16
    acc = pl.pallas_call(
        _conv_slices_body,
        grid=(B // bb,),
        in_specs=[
            pl.BlockSpec((bb, 128, 8 * DIM), lambda i: (i, 0, 0)),
            pl.BlockSpec((1, DIM), lambda i: (0, 0)),
        ],
        out_specs=pl.BlockSpec((bb, 121, DIM), lambda i: (i, 0, 0)),
        out_shape=jax.ShapeDtypeStruct((B, 121, DIM), _f32),
    )(p3, cb)
    flat = acc.reshape(B, 121 * DIM)
    return _pc(_mm_bias_body, jax.ShapeDtypeStruct((B, 128), _f32), flat, wperm, fb)


def _final_body(h, bt, xdo, fw, fb, w1, b1, w2, b2, w3, b3, pooled, out):
    i = pl.program_id(0)

    @pl.when(i == 0)
    def _():
        pooled[...] = jnp.zeros_like(pooled)

    bb = bt[0, 0, :]
    ohot = (bb[:, None] == lax.broadcasted_iota(jnp.int32, (R, B), 1)).astype(_f32)
    pooled[...] += lax.dot_general(ohot, h[...], (((0,), (0,)), ((), ())),
                                   preferred_element_type=_f32)

    @pl.when(i == NBLK - 1)
    def _():
        xt_out = jnp.maximum(
            jnp.dot(pooled[...], fw[...], preferred_element_type=_f32) + fb[...], 0.0)
        xj = jnp.concatenate([xdo[...], xt_out], axis=1)
        z = jnp.maximum(jnp.dot(xj, w1[...], preferred_element_type=_f32) + b1[...], 0.0)
        z = jnp.maximum(jnp.dot(z, w2[...], preferred_element_type=_f32) + b2[...], 0.0)
        out[...] = jnp.dot(z, w3[...], preferred_element_type=_f32) + b3[...]


def _final(h, batch3, xdo, fw, fb, w1, b1, w2, b2, w3, b3):
    pooled, out = pl.pallas_call(
        _final_body,
        grid=(NBLK,),
        in_specs=[
            pl.BlockSpec((R, DIM), lambda i: (i, 0)),
            pl.BlockSpec((1, 1, R), lambda i: (i, 0, 0)),
            pl.BlockSpec((B, 128), lambda i: (0, 0)),
            pl.BlockSpec((DIM, 128), lambda i: (0, 0)),
            pl.BlockSpec((1, 128), lambda i: (0, 0)),
            pl.BlockSpec((256, 1024), lambda i: (0, 0)),
            pl.BlockSpec((1, 1024), lambda i: (0, 0)),
            pl.BlockSpec((1024, 256), lambda i: (0, 0)),
            pl.BlockSpec((1, 256), lambda i: (0, 0)),
            pl.BlockSpec((256, 1), lambda i: (0, 0)),
            pl.BlockSpec((1, 1), lambda i: (0, 0)),
        ],
        out_specs=[
            pl.BlockSpec((B, DIM), lambda i: (0, 0)),
            pl.BlockSpec((B, 1), lambda i: (0, 0)),
        ],
        out_shape=[
            jax.ShapeDtypeStruct((B, DIM), _f32),
            jax.ShapeDtypeStruct((B, 1), _f32),
        ],
    )(h, batch3, xdo, fw, fb, w1, b1, w2, b2, w3, b3)
    return out


def kernel(xd, xt, xt_edge_index, xt_batch, y, params):
    p = params
    bn_scale = jnp.float32(1.0 / jnp.sqrt(1.0 + 1e-5))

    wr2 = p["conv_w"].transpose(1, 2, 0).reshape(100, 8 * DIM)
    wperm = p["fc1_xd_w"].reshape(DIM, 121, 128).transpose(1, 0, 2).reshape(121 * DIM, 128)
    xd_out = _drug(xd, p["emb_xd"], wr2, p["conv_b"].reshape(1, DIM), wperm,
                   p["fc1_xd_b"].reshape(1, 128))

    edges = xt_edge_index.astype(jnp.int32)
    qlo, qhi = _pre(xt, p["gin"][0]["w1"])
    h = None
    agg = _make_agg()
    for k in range(5):
        s2 = agg(qlo, qhi, edges)
        gp, bp = p["gin"][k], p["bn"][k]
        b1 = gp["b1"].reshape(1, DIM)
        w2, b2 = gp["w2"], gp["b2"].reshape(1, DIM)
        gsc = (bp["g"] * bn_scale).reshape(1, DIM)
        gb = bp["b"].reshape(1, DIM)
        if k < 4:
            qlo, qhi = _layer(qlo, qhi, s2, b1, w2, b2, gsc, gb,
                              p["gin"][k + 1]["w1"])
        else:
            h = _last(qlo, qhi, s2, b1, w2, b2, gsc, gb)

    batch3 = xt_batch.astype(jnp.int32).reshape(NBLK, 1, R)
    out = _final(h, batch3, xd_out, p["fc1_xt_w"], p["fc1_xt_b"].reshape(1, 128),
                 p["cls_w1"], p["cls_b1"].reshape(1, 1024),
                 p["cls_w2"], p["cls_b2"].reshape(1, 256),
                 p["cls_w3"], p["cls_b3"].reshape(1, 1))
    return (out.reshape(B), y)


# speedup vs baseline: 16.5490x; 1.0399x over previous
import functools

import jax
import jax.numpy as jnp
from jax import lax
from jax.experimental import pallas as pl
from jax.experimental.pallas import tpu as pltpu
from jax.experimental.pallas import tpu_sc as plsc

N = 100000
E = 1600000
B = 128
DIM = 32
HALF = 16

NC, NS, L = 2, 16, 16
W = 128
SCW = 256
NSUP = E // SCW
SUP_BASE = NSUP // NS
SUP_REM = NSUP % NS
NBUF = 6
NGRP = -(-(SUP_BASE + 1) // NBUF)
ZB = 128
DR = 6144
NZCH = DR // ZB
DR_LAST = N - (NS - 1) * DR
NZCH_LAST = DR_LAST // ZB
ZTAIL = DR_LAST - NZCH_LAST * ZB

_f32 = jnp.float32


def _agg_body(qlo, qhi, edges, out, acc, zbuf, idx, rows, isem, gsem, ssem):
    c = lax.axis_index("c")
    s = lax.axis_index("s")

    def _zrow(i, carry):
        zbuf[i, :] = jnp.zeros((L,), _f32)
        return carry
    lax.fori_loop(0, ZB, _zrow, 0)
    rbase = pl.multiple_of(s * DR, 8)

    for t in range(NZCH):
        pltpu.sync_copy(zbuf, acc.at[pl.ds(rbase + t * ZB, ZB)])

    @pl.when(s == NS - 1)
    def _():
        for t in range(NZCH, NZCH_LAST):
            pltpu.sync_copy(zbuf, acc.at[pl.ds(rbase + t * ZB, ZB)])
        pltpu.sync_copy(zbuf.at[pl.ds(0, ZTAIL)],
                        acc.at[pl.ds(rbase + NZCH_LAST * ZB, ZTAIL)])
    plsc.subcore_barrier()

    start = s * SUP_BASE + jnp.minimum(s, SUP_REM)
    end = start + SUP_BASE + jnp.where(s < SUP_REM, 1, 0)

    def _chunk_ops(b, ch):
        off = pl.multiple_of(ch * SCW, SCW)
        def idx_load():
            pltpu.async_copy(edges.at[:, pl.ds(off, SCW)], idx.at[b], isem.at[b])
        def gather_start():
            pltpu.make_async_copy(edges.at[:, pl.ds(off, SCW)], idx.at[b],
                                  isem.at[b]).wait()
            src = idx.at[b, 0]
            @pl.when(c == 0)
            def _():
                pltpu.async_copy(qlo.at[src], rows.at[b], gsem.at[b])
            @pl.when(c == 1)
            def _():
                pltpu.async_copy(qhi.at[src], rows.at[b], gsem.at[b])
        def gather_wait_scatter_start():
            pltpu.make_async_copy(qlo.at[idx.at[b, 0]], rows.at[b],
                                  gsem.at[b]).wait()
            pltpu.async_copy(rows.at[b], acc.at[idx.at[b, 1]], ssem.at[b],
                             add=True)
        def scatter_wait():
            pltpu.make_async_copy(rows.at[b], acc.at[idx.at[b, 1]],
                                  ssem.at[b]).wait()
        return idx_load, gather_start, gather_wait_scatter_start, scatter_wait

    def _group(g, carry):
        ch0 = start + g * NBUF
        ops = [_chunk_ops(b, ch0 + b) for b in range(NBUF)]
        for b in range(NBUF):
            @pl.when(g > 0)
            def _(b=b):
                ops[b][3]()
        for b in range(NBUF):
            @pl.when(ch0 + b < end)
            def _(b=b):
                ops[b][0]()
        for b in range(NBUF):
            @pl.when(ch0 + b < end)
            def _(b=b):
                ops[b][1]()
        for b in range(NBUF):
            @pl.when(ch0 + b < end)
            def _(b=b):
                ops[b][2]()
        return carry
    lax.fori_loop(0, NGRP, _group, 0)

    chf = start + (NGRP - 1) * NBUF
    for b in range(NBUF):
        ops = _chunk_ops(b, chf + b)
        @pl.when(chf + b < end)
        def _(sw=ops[3]):
            sw()

    plsc.subcore_barrier()

    @pl.when(s < NS - 1)
    def _():
        pltpu.sync_copy(acc.at[pl.ds(rbase, DR)], out.at[c, pl.ds(rbase, DR)])

    @pl.when(s == NS - 1)
    def _():
        pltpu.sync_copy(acc.at[pl.ds(rbase, DR_LAST)],
                        out.at[c, pl.ds(rbase, DR_LAST)])


@functools.cache
def _make_agg():
  return pl.kernel(
    _agg_body,
    out_type=jax.ShapeDtypeStruct((NC, N, HALF), _f32),
    mesh=plsc.VectorSubcoreMesh(core_axis_name="c", subcore_axis_name="s",
                                num_cores=NC, num_subcores=NS),
    compiler_params=pltpu.CompilerParams(use_tc_tiling_on_sc=False),
    scratch_types=[
        pltpu.VMEM_SHARED((N, HALF), _f32),
        pltpu.VMEM((ZB, L), _f32),
        pltpu.VMEM((NBUF, 2, SCW), jnp.int32),
        pltpu.VMEM((NBUF, SCW, HALF), _f32),
        pltpu.SemaphoreType.DMA((NBUF,)),
        pltpu.SemaphoreType.DMA((NBUF,)),
        pltpu.SemaphoreType.DMA((NBUF,)),
    ],
  )


R = 5000
NBLK = N // R


def _pre_body(xt, w1, olo, ohi):
    q = jnp.dot(xt[...], w1[...], preferred_element_type=_f32)
    olo[...] = q[:, :HALF]
    ohi[...] = q[:, HALF:]


def _pre(xt, w1):
    return pl.pallas_call(
        _pre_body,
        grid=(NBLK,),
        in_specs=[
            pl.BlockSpec((R, 41), lambda i: (i, 0)),
            pl.BlockSpec((41, DIM), lambda i: (0, 0)),
        ],
        out_specs=[pl.BlockSpec((R, HALF), lambda i: (i, 0))] * 2,
        out_shape=[jax.ShapeDtypeStruct((N, HALF), _f32)] * 2,
    )(xt, w1)


def _layer_body(qlo, qhi, slo, shi, b1, w2, b2, gsc, gb, w1n, olo, ohi):
    q = jnp.concatenate([qlo[...], qhi[...]], axis=1)
    sagg = jnp.concatenate([slo[0], shi[0]], axis=1)
    z = jnp.maximum(q + sagg + b1[...], 0.0)
    z = jnp.maximum(jnp.dot(z, w2[...], preferred_element_type=_f32) + b2[...], 0.0)
    h = z * gsc[...] + gb[...]
    qn = jnp.dot(h, w1n[...], preferred_element_type=_f32)
    olo[...] = qn[:, :HALF]
    ohi[...] = qn[:, HALF:]


def _layer(qlo, qhi, s2, b1, w2, b2, gsc, gb, w1n):
    return pl.pallas_call(
        _layer_body,
        grid=(NBLK,),
        in_specs=[
            pl.BlockSpec((R, HALF), lambda i: (i, 0)),
            pl.BlockSpec((R, HALF), lambda i: (i, 0)),
            pl.BlockSpec((1, R, HALF), lambda i: (0, i, 0)),
            pl.BlockSpec((1, R, HALF), lambda i: (1, i, 0)),
            pl.BlockSpec((1, DIM), lambda i: (0, 0)),
            pl.BlockSpec((DIM, DIM), lambda i: (0, 0)),
            pl.BlockSpec((1, DIM), lambda i: (0, 0)),
            pl.BlockSpec((1, DIM), lambda i: (0, 0)),
            pl.BlockSpec((1, DIM), lambda i: (0, 0)),
            pl.BlockSpec((DIM, DIM), lambda i: (0, 0)),
        ],
        out_specs=[pl.BlockSpec((R, HALF), lambda i: (i, 0))] * 2,
        out_shape=[jax.ShapeDtypeStruct((N, HALF), _f32)] * 2,
    )(qlo, qhi, s2, s2, b1, w2, b2, gsc, gb, w1n)


def _last_body(qlo, qhi, slo, shi, b1, w2, b2, gsc, gb, oh):
    q = jnp.concatenate([qlo[...], qhi[...]], axis=1)
    sagg = jnp.concatenate([slo[0], shi[0]], axis=1)
    z = jnp.maximum(q + sagg + b1[...], 0.0)
    z = jnp.maximum(jnp.dot(z, w2[...], preferred_element_type=_f32) + b2[...], 0.0)
    oh[...] = z * gsc[...] + gb[...]


def _last(qlo, qhi, s2, b1, w2, b2, gsc, gb):
    return pl.pallas_call(
        _last_body,
        grid=(NBLK,),
        in_specs=[
            pl.BlockSpec((R, HALF), lambda i: (i, 0)),
            pl.BlockSpec((R, HALF), lambda i: (i, 0)),
            pl.BlockSpec((1, R, HALF), lambda i: (0, i, 0)),
            pl.BlockSpec((1, R, HALF), lambda i: (1, i, 0)),
            pl.BlockSpec((1, DIM), lambda i: (0, 0)),
            pl.BlockSpec((DIM, DIM), lambda i: (0, 0)),
            pl.BlockSpec((1, DIM), lambda i: (0, 0)),
            pl.BlockSpec((1, DIM), lambda i: (0, 0)),
            pl.BlockSpec((1, DIM), lambda i: (0, 0)),
        ],
        out_specs=pl.BlockSpec((R, DIM), lambda i: (i, 0)),
        out_shape=jax.ShapeDtypeStruct((N, DIM), _f32),
    )(qlo, qhi, s2, s2, b1, w2, b2, gsc, gb)


def _emb_body(xdF, table, out):
    ohot = (xdF[...] == lax.broadcasted_iota(jnp.int32, (100 * B, 65), 1)).astype(_f32)
    out[...] = jnp.dot(ohot, table[...], preferred_element_type=_f32)


def _mm_body(a, b, out):
    out[...] = jnp.dot(a[...], b[...], preferred_element_type=_f32)


def _conv_slices_body(p3, cb, out):
    acc = p3[:, 0:121, 0:DIM]
    for k in range(1, 8):
        acc = acc + p3[:, k:k + 121, k * DIM:(k + 1) * DIM]
    out[...] = acc + cb[...].reshape(1, 1, DIM)


def _mm_bias_body(a, b, bias, out):
    out[...] = jnp.dot(a[...], b[...], preferred_element_type=_f32) + bias[...]


def _pc(body, out_shape, *args):
    return pl.pallas_call(body, out_shape=out_shape)(*args)


def _drug(xd, table, wr2, cb, wperm, fb):
    xdF = xd.T.reshape(100 * B, 1)
    emb3 = _pc(_emb_body, jax.ShapeDtypeStruct((100 * B, 128), _f32), xdF, table)
    at = emb3.reshape(100, B * 128).T
    p2 = _pc(_mm_body, jax.ShapeDtypeStruct((B * 128, 8 * DIM), _f32), at, wr2)
    p3 = p2.reshape(B, 128, 8 * DIM)
    bb = 16
    acc = pl.pallas_call(
        _conv_slices_body,
        grid=(B // bb,),
        in_specs=[
            pl.BlockSpec((bb, 128, 8 * DIM), lambda i: (i, 0, 0)),
            pl.BlockSpec((1, DIM), lambda i: (0, 0)),
        ],
        out_specs=pl.BlockSpec((bb, 121, DIM), lambda i: (i, 0, 0)),
        out_shape=jax.ShapeDtypeStruct((B, 121, DIM), _f32),
    )(p3, cb)
    flat = acc.reshape(B, 121 * DIM)
    return _pc(_mm_bias_body, jax.ShapeDtypeStruct((B, 128), _f32), flat, wperm, fb)


def _final_body(h, bt, xdo, fw, fb, w1, b1, w2, b2, w3, b3, pooled, out):
    i = pl.program_id(0)

    @pl.when(i == 0)
    def _():
        pooled[...] = jnp.zeros_like(pooled)

    bb = bt[0, 0, :]
    ohot = (bb[:, None] == lax.broadcasted_iota(jnp.int32, (R, B), 1)).astype(_f32)
    pooled[...] += lax.dot_general(ohot, h[...], (((0,), (0,)), ((), ())),
                                   preferred_element_type=_f32)

    @pl.when(i == NBLK - 1)
    def _():
        xt_out = jnp.maximum(
            jnp.dot(pooled[...], fw[...], preferred_element_type=_f32) + fb[...], 0.0)
        xj = jnp.concatenate([xdo[...], xt_out], axis=1)
        z = jnp.maximum(jnp.dot(xj, w1[...], preferred_element_type=_f32) + b1[...], 0.0)
        z = jnp.maximum(jnp.dot(z, w2[...], preferred_element_type=_f32) + b2[...], 0.0)
        out[...] = jnp.dot(z, w3[...], preferred_element_type=_f32) + b3[...]


def _final(h, batch3, xdo, fw, fb, w1, b1, w2, b2, w3, b3):
    pooled, out = pl.pallas_call(
        _final_body,
        grid=(NBLK,),
        in_specs=[
            pl.BlockSpec((R, DIM), lambda i: (i, 0)),
            pl.BlockSpec((1, 1, R), lambda i: (i, 0, 0)),
            pl.BlockSpec((B, 128), lambda i: (0, 0)),
            pl.BlockSpec((DIM, 128), lambda i: (0, 0)),
            pl.BlockSpec((1, 128), lambda i: (0, 0)),
            pl.BlockSpec((256, 1024), lambda i: (0, 0)),
            pl.BlockSpec((1, 1024), lambda i: (0, 0)),
            pl.BlockSpec((1024, 256), lambda i: (0, 0)),
            pl.BlockSpec((1, 256), lambda i: (0, 0)),
            pl.BlockSpec((256, 1), lambda i: (0, 0)),
            pl.BlockSpec((1, 1), lambda i: (0, 0)),
        ],
        out_specs=[
            pl.BlockSpec((B, DIM), lambda i: (0, 0)),
            pl.BlockSpec((B, 1), lambda i: (0, 0)),
        ],
        out_shape=[
            jax.ShapeDtypeStruct((B, DIM), _f32),
            jax.ShapeDtypeStruct((B, 1), _f32),
        ],
    )(h, batch3, xdo, fw, fb, w1, b1, w2, b2, w3, b3)
    return out


def kernel(xd, xt, xt_edge_index, xt_batch, y, params):
    p = params
    bn_scale = jnp.float32(1.0 / jnp.sqrt(1.0 + 1e-5))

    wr2 = p["conv_w"].transpose(1, 2, 0).reshape(100, 8 * DIM)
    wperm = p["fc1_xd_w"].reshape(DIM, 121, 128).transpose(1, 0, 2).reshape(121 * DIM, 128)
    xd_out = _drug(xd, p["emb_xd"], wr2, p["conv_b"].reshape(1, DIM), wperm,
                   p["fc1_xd_b"].reshape(1, 128))

    edges = xt_edge_index.astype(jnp.int32)
    qlo, qhi = _pre(xt, p["gin"][0]["w1"])
    h = None
    agg = _make_agg()
    for k in range(5):
        s2 = agg(qlo, qhi, edges)
        gp, bp = p["gin"][k], p["bn"][k]
        b1 = gp["b1"].reshape(1, DIM)
        w2, b2 = gp["w2"], gp["b2"].reshape(1, DIM)
        gsc = (bp["g"] * bn_scale).reshape(1, DIM)
        gb = bp["b"].reshape(1, DIM)
        if k < 4:
            qlo, qhi = _layer(qlo, qhi, s2, b1, w2, b2, gsc, gb,
                              p["gin"][k + 1]["w1"])
        else:
            h = _last(qlo, qhi, s2, b1, w2, b2, gsc, gb)

    batch3 = xt_batch.astype(jnp.int32).reshape(NBLK, 1, R)
    out = _final(h, batch3, xd_out, p["fc1_xt_w"], p["fc1_xt_b"].reshape(1, 128),
                 p["cls_w1"], p["cls_b1"].reshape(1, 1024),
                 p["cls_w2"], p["cls_b2"].reshape(1, 256),
                 p["cls_w3"], p["cls_b3"].reshape(1, 1))
    return (out.reshape(B), y)
